# baseline (speedup 1.0000x reference)
import functools

import jax
import jax.numpy as jnp
from jax import lax
from jax.experimental import pallas as pl
from jax.experimental.pallas import tpu as pltpu
from jax.experimental.pallas import tpu_sc as plsc

ROWS = 128
N = 32768
KK = 256
L = 16
NCHUNK = N // L
NWORKERS = 32
ROWS_PER_W = ROWS // NWORKERS
FBINS = 1024
NGROUPS = FBINS // L
DUMMY = NCHUNK

MIN32 = -(2 ** 31)
M7F = 0x7FFFFFFF
LOW22 = 0x3FFFFF


def _key_of(v):
    b = lax.bitcast_convert_type(v, jnp.int32)
    return b ^ ((b >> 31) & M7F)


def _topk_body(x_hbm, out_hbm, row_v, out_v, fine_v, gsum_v, ck_v, cmax_v,
               list_v):
    cid = lax.axis_index("c")
    sid = lax.axis_index("s")
    wid = sid * 2 + cid

    zeros16 = jnp.zeros((L,), jnp.int32)
    zerosf16 = jnp.zeros((L,), jnp.float32)
    ones16 = jnp.ones((L,), jnp.int32)
    iota16 = lax.iota(jnp.int32, L)
    lane_off = iota16 * FBINS
    lane15 = iota16 == (L - 1)

    row_v[pl.ds(N, L)] = jnp.full((L,), -jnp.inf, jnp.float32)

    def per_row(r, _carry):
        row = wid * ROWS_PER_W + r
        pltpu.sync_copy(x_hbm.at[row], row_v.at[pl.ds(0, N)])

        @plsc.parallel_loop(0, FBINS, unroll=8)
        def _clear(i):
            fine_v[pl.ds(pl.multiple_of(i * L, L), L)] = zeros16

        @plsc.parallel_loop(0, NCHUNK, unroll=8)
        def _hist(i):
            v = row_v[pl.ds(pl.multiple_of(i * L, L), L)]
            key = _key_of(v)
            ubin = ((key ^ MIN32) >> 22) & (FBINS - 1)
            plsc.addupdate_scatter(fine_v, [lane_off + ubin], ones16)
            cmx = plsc.cummax(ubin)
            plsc.store_scatter(cmax_v, [jnp.full((L,), i, jnp.int32)], cmx,
                               mask=lane15)

        @plsc.parallel_loop(0, NGROUPS, unroll=2)
        def _gsum(g):
            base = g * L
            vec = fine_v[pl.ds(pl.multiple_of(base, L), L)]
            for l in range(1, L):
                vec = vec + fine_v[pl.ds(pl.multiple_of(l * FBINS + base, L), L)]
            cum = plsc.cumsum(vec)
            gidx = jnp.full((L,), g, jnp.int32)
            plsc.store_scatter(gsum_v, [gidx], cum, mask=lane15)

        gvecs = [gsum_v[pl.ds(k * L, L)] for k in range(NGROUPS // L)]
        sufs = [jnp.flip(plsc.cumsum(jnp.flip(g, 0)), 0) for g in gvecs]
        tots = [s[0] for s in sufs]
        cnt_g = zeros16
        hi = jnp.int32(0)
        for k in range(NGROUPS // L - 1, -1, -1):
            cnt_g = cnt_g + plsc.all_reduce_population_count((sufs[k] + hi) >= KK)
            hi = hi + tots[k]
        gstar = jnp.max(cnt_g) - 1
        acc_above = jnp.int32(0)
        for k in range(NGROUPS // L):
            acc_above = acc_above + jnp.sum(
                jnp.where((k * L + iota16) > gstar, gvecs[k], 0))

        base = gstar * L
        vec = fine_v[pl.ds(pl.multiple_of(base, L), L)]
        for l in range(1, L):
            vec = vec + fine_v[pl.ds(pl.multiple_of(l * FBINS + base, L), L)]
        suf = jnp.flip(plsc.cumsum(jnp.flip(vec, 0)), 0)
        cond = (acc_above + suf) >= KK
        cntv = plsc.all_reduce_population_count(cond)
        posv = cntv - 1
        b1v = gstar * L + posv
        count_above = acc_above + jnp.sum(jnp.where(iota16 > posv, vec, 0))
        k_rem = jnp.int32(KK) - count_above
        b1s = gstar * L + jnp.max(cntv) - 1

        @plsc.parallel_loop(0, NCHUNK // L, unroll=4, carry=zeros16)
        def _mklist(j, nl_vec):
            cm = cmax_v[pl.ds(pl.multiple_of(j * L, L), L)]
            m = cm >= b1s
            pref = plsc.cumsum(m.astype(jnp.int32))
            pos = nl_vec + pref - 1
            plsc.store_scatter(list_v, [pos], j * L + iota16, mask=m)
            return nl_vec + plsc.all_reduce_population_count(m)

        nl = jnp.max(_mklist)
        nlch = (nl + (L - 1)) // L
        tbase = pl.multiple_of((nlch - 1) * L, L)
        tidx = (nlch - 1) * L + iota16
        tvec = list_v[pl.ds(tbase, L)]
        list_v[pl.ds(tbase, L)] = jnp.where(tidx < nl, tvec, DUMMY)

        def comp_group(j, nc_vec):
            lvec = list_v[pl.ds(pl.multiple_of(j * L, L), L)]
            for l in range(L):
                cb = pl.multiple_of(lvec[l] * L, L)
                v = row_v[pl.ds(cb, L)]
                key = _key_of(v)
                ux = key ^ MIN32
                ubin = (ux >> 22) & (FBINS - 1)
                m = jnp.logical_and(ubin == b1v, (j * L + l) < nl)
                pref = plsc.cumsum(m.astype(jnp.int32))
                pos = nc_vec + pref - 1
                plsc.store_scatter(ck_v, [pos], ux & LOW22, mask=m)
                nc_vec = nc_vec + plsc.all_reduce_population_count(m)
            return nc_vec

        nc = jnp.max(lax.fori_loop(0, nlch, comp_group, zeros16))
        nch = (nc + (L - 1)) // L

        def refine(bi, prefix):
            bit = jnp.int32(21) - bi
            cand = prefix | (jnp.int32(1) << bit)

            def cnt_chunk(j, cnt_v):
                low = ck_v[pl.ds(pl.multiple_of(j * L, L), L)]
                valid = (j * L + iota16) < nc
                m = jnp.logical_and(low >= cand, valid)
                return cnt_v + plsc.all_reduce_population_count(m)

            cnt = jnp.max(lax.fori_loop(0, nch, cnt_chunk, zeros16))
            return jnp.where(cnt >= k_rem, cand, prefix)

        prefix = lax.fori_loop(0, 22, refine, jnp.int32(0))

        def gt_chunk(j, cnt_v):
            low = ck_v[pl.ds(pl.multiple_of(j * L, L), L)]
            valid = (j * L + iota16) < nc
            m = jnp.logical_and(low > prefix, valid)
            return cnt_v + plsc.all_reduce_population_count(m)

        c_gt = jnp.max(lax.fori_loop(0, nch, gt_chunk, zeros16))
        ties_needed = k_rem - c_gt

        t_u = (b1s << 22) | prefix
        t_s = t_u ^ MIN32
        tb_vec = jnp.full((L,), t_s, jnp.int32)
        tb_vec = tb_vec ^ ((tb_vec >> 31) & M7F)
        t_f_vec = lax.bitcast_convert_type(tb_vec, jnp.float32)

        @plsc.parallel_loop(0, NCHUNK + 1, unroll=8)
        def _zfill(i):
            out_v[pl.ds(pl.multiple_of(i * L, L), L)] = zerosf16

        @plsc.parallel_loop(0, nlch, unroll=1)
        def _sel(j):
            lvec = list_v[pl.ds(pl.multiple_of(j * L, L), L)]
            for l in range(L):
                cb = pl.multiple_of(lvec[l] * L, L)
                v = row_v[pl.ds(cb, L)]
                key = _key_of(v)
                out_v[pl.ds(cb, L)] = jnp.where(key > t_s, v, jnp.float32(0.0))

        def fixup(j, seq_vec):
            lvec = list_v[pl.ds(pl.multiple_of(j * L, L), L)]
            for l in range(L):
                cs = lvec[l]
                v = row_v[pl.ds(pl.multiple_of(cs * L, L), L)]
                key = _key_of(v)
                m_eq = jnp.logical_and(key == t_s, (j * L + l) < nl)
                pref = plsc.cumsum(m_eq.astype(jnp.int32))
                keep = jnp.logical_and(m_eq, (seq_vec + pref) <= ties_needed)
                plsc.store_scatter(out_v, [cs * L + iota16], t_f_vec, mask=keep)
                seq_vec = seq_vec + plsc.all_reduce_population_count(m_eq)
            return seq_vec

        lax.fori_loop(0, nlch, fixup, zeros16)

        pltpu.sync_copy(out_v.at[pl.ds(0, N)], out_hbm.at[row])
        return _carry

    lax.fori_loop(0, ROWS_PER_W, per_row, 0)


@functools.partial(
    pl.kernel,
    out_type=jax.ShapeDtypeStruct((ROWS, N), jnp.float32),
    mesh=plsc.VectorSubcoreMesh(core_axis_name="c", subcore_axis_name="s"),
    compiler_params=pltpu.CompilerParams(needs_layout_passes=False),
    scratch_types=[
        pltpu.VMEM((N + L,), jnp.float32),
        pltpu.VMEM((N + L,), jnp.float32),
        pltpu.VMEM((L * FBINS,), jnp.int32),
        pltpu.VMEM((NGROUPS,), jnp.int32),
        pltpu.VMEM((N,), jnp.int32),
        pltpu.VMEM((NCHUNK,), jnp.int32),
        pltpu.VMEM((NCHUNK,), jnp.int32),
    ],
)
def _topk_sc(x_hbm, out_hbm, row_v, out_v, fine_v, gsum_v, ck_v, cmax_v,
             list_v):
    _topk_body(x_hbm, out_hbm, row_v, out_v, fine_v, gsum_v, ck_v, cmax_v,
               list_v)


def kernel(x):
    return _topk_sc(x)


# speedup vs baseline: 1.5887x; 1.5887x over previous
import functools

import jax
import jax.numpy as jnp
from jax import lax
from jax.experimental import pallas as pl
from jax.experimental.pallas import tpu as pltpu
from jax.experimental.pallas import tpu_sc as plsc

ROWS = 128
N = 32768
KK = 256
L = 16
NCHUNK = N // L
NWORKERS = 32
ROWS_PER_W = ROWS // NWORKERS
FBINS = 1024
NGROUPS = FBINS // L

MIN32 = -(2 ** 31)
M7F = 0x7FFFFFFF
LOW22 = 0x3FFFFF


def _key_of(v):
    b = lax.bitcast_convert_type(v, jnp.int32)
    return b ^ ((b >> 31) & M7F)


def _topk_body(x_hbm, out_hbm, row_a, row_b, fine_v, gsum_v, ci_v,
               sem_ia, sem_ib, sem_oa, sem_ob):
    cid = lax.axis_index("c")
    sid = lax.axis_index("s")
    wid = sid * 2 + cid

    zeros16 = jnp.zeros((L,), jnp.int32)
    zerosf16 = jnp.zeros((L,), jnp.float32)
    ones16 = jnp.ones((L,), jnp.int32)
    iota16 = lax.iota(jnp.int32, L)
    lane_off = iota16 * FBINS
    lane15 = iota16 == (L - 1)

    bufs = [row_a, row_b]
    in_sems = [sem_ia, sem_ib]
    out_sems = [sem_oa, sem_ob]

    def compute(row_v, hook):
        @plsc.parallel_loop(0, FBINS, unroll=8)
        def _clear(i):
            fine_v[pl.ds(pl.multiple_of(i * L, L), L)] = zeros16

        @plsc.parallel_loop(0, NCHUNK, unroll=8)
        def _hist(i):
            v = row_v[pl.ds(pl.multiple_of(i * L, L), L)]
            key = _key_of(v)
            ubin = ((key ^ MIN32) >> 22) & (FBINS - 1)
            plsc.addupdate_scatter(fine_v, [lane_off + ubin], ones16)

        hook()

        @plsc.parallel_loop(0, NGROUPS, unroll=2)
        def _gsum(g):
            base = g * L
            vec = fine_v[pl.ds(pl.multiple_of(base, L), L)]
            for l in range(1, L):
                vec = vec + fine_v[pl.ds(pl.multiple_of(l * FBINS + base, L), L)]
            cum = plsc.cumsum(vec)
            gidx = jnp.full((L,), g, jnp.int32)
            plsc.store_scatter(gsum_v, [gidx], cum, mask=lane15)

        gvecs = [gsum_v[pl.ds(k * L, L)] for k in range(NGROUPS // L)]
        sufs = [jnp.flip(plsc.cumsum(jnp.flip(g, 0)), 0) for g in gvecs]
        tots = [s[0] for s in sufs]
        cnt_g = zeros16
        hi = jnp.int32(0)
        for k in range(NGROUPS // L - 1, -1, -1):
            cnt_g = cnt_g + plsc.all_reduce_population_count((sufs[k] + hi) >= KK)
            hi = hi + tots[k]
        gstar = jnp.max(cnt_g) - 1
        acc_above = jnp.int32(0)
        for k in range(NGROUPS // L):
            acc_above = acc_above + jnp.sum(
                jnp.where((k * L + iota16) > gstar, gvecs[k], 0))

        base = gstar * L
        vec = fine_v[pl.ds(pl.multiple_of(base, L), L)]
        for l in range(1, L):
            vec = vec + fine_v[pl.ds(pl.multiple_of(l * FBINS + base, L), L)]
        suf = jnp.flip(plsc.cumsum(jnp.flip(vec, 0)), 0)
        cond = (acc_above + suf) >= KK
        cntv = plsc.all_reduce_population_count(cond)
        posv = cntv - 1
        b1v = gstar * L + posv
        count_above = acc_above + jnp.sum(jnp.where(iota16 > posv, vec, 0))
        k_rem = jnp.int32(KK) - count_above
        b1s = gstar * L + jnp.max(cntv) - 1

        @plsc.parallel_loop(0, NCHUNK, unroll=4, carry=zeros16)
        def _comp(i, nc_vec):
            v = row_v[pl.ds(pl.multiple_of(i * L, L), L)]
            key = _key_of(v)
            ubin = ((key ^ MIN32) >> 22) & (FBINS - 1)
            m = ubin == b1v
            pref = plsc.cumsum(m.astype(jnp.int32))
            pos = nc_vec + pref - 1
            plsc.store_scatter(ci_v, [pos], i * L + iota16, mask=m)
            return nc_vec + plsc.all_reduce_population_count(m)

        nc = jnp.max(_comp)
        nch = (nc + (L - 1)) // L

        def cand_low(j, mask):
            idx = ci_v[pl.ds(pl.multiple_of(j * L, L), L)]
            v = plsc.load_gather(row_v, [idx], mask=mask)
            return (_key_of(v) ^ MIN32) & LOW22

        def refine(bi, prefix):
            bit = jnp.int32(21) - bi
            cand = prefix | (jnp.int32(1) << bit)

            def cnt_chunk(j, cnt_v):
                valid = (j * L + iota16) < nc
                m = jnp.logical_and(cand_low(j, valid) >= cand, valid)
                return cnt_v + plsc.all_reduce_population_count(m)

            cnt = jnp.max(lax.fori_loop(0, nch, cnt_chunk, zeros16))
            return jnp.where(cnt >= k_rem, cand, prefix)

        prefix = lax.fori_loop(0, 22, refine, jnp.int32(0))

        def gt_chunk(j, cnt_v):
            valid = (j * L + iota16) < nc
            m = jnp.logical_and(cand_low(j, valid) > prefix, valid)
            return cnt_v + plsc.all_reduce_population_count(m)

        c_gt = jnp.max(lax.fori_loop(0, nch, gt_chunk, zeros16))
        ties_needed = k_rem - c_gt

        t_u = (b1s << 22) | prefix
        t_s = t_u ^ MIN32

        @plsc.parallel_loop(0, NCHUNK, unroll=8)
        def _outp(i):
            sl = pl.ds(pl.multiple_of(i * L, L), L)
            v = row_v[sl]
            key = _key_of(v)
            row_v[sl] = jnp.where(key >= t_s, v, jnp.float32(0.0))

        def fixup(j, seq_vec):
            valid = (j * L + iota16) < nc
            idx = ci_v[pl.ds(pl.multiple_of(j * L, L), L)]
            m_eq = jnp.logical_and(cand_low(j, valid) == prefix, valid)
            pref = plsc.cumsum(m_eq.astype(jnp.int32))
            drop = jnp.logical_and(m_eq, (seq_vec + pref) > ties_needed)
            plsc.store_scatter(row_v, [idx], zerosf16, mask=drop)
            return seq_vec + plsc.all_reduce_population_count(m_eq)

        lax.fori_loop(0, nch, fixup, zeros16)

    def start_in(r):
        return pltpu.async_copy(x_hbm.at[wid * ROWS_PER_W + r], bufs[r % 2],
                                in_sems[r % 2])

    def start_out(r):
        return pltpu.async_copy(bufs[r % 2], out_hbm.at[wid * ROWS_PER_W + r],
                                out_sems[r % 2])

    in_d = {0: start_in(0), 1: start_in(1)}
    out_d = {}
    for r in range(ROWS_PER_W):
        in_d[r].wait()

        def hook(r=r):
            if r >= 1 and r + 1 < ROWS_PER_W:
                out_d[r - 1].wait()
                in_d[r + 1] = start_in(r + 1)

        compute(bufs[r % 2], hook)
        out_d[r] = start_out(r)
    out_d[ROWS_PER_W - 2].wait()
    out_d[ROWS_PER_W - 1].wait()


@functools.partial(
    pl.kernel,
    out_type=jax.ShapeDtypeStruct((ROWS, N), jnp.float32),
    mesh=plsc.VectorSubcoreMesh(core_axis_name="c", subcore_axis_name="s"),
    compiler_params=pltpu.CompilerParams(needs_layout_passes=False),
    scratch_types=[
        pltpu.VMEM((N,), jnp.float32),
        pltpu.VMEM((N,), jnp.float32),
        pltpu.VMEM((L * FBINS,), jnp.int32),
        pltpu.VMEM((NGROUPS,), jnp.int32),
        pltpu.VMEM((N,), jnp.int32),
        pltpu.SemaphoreType.DMA,
        pltpu.SemaphoreType.DMA,
        pltpu.SemaphoreType.DMA,
        pltpu.SemaphoreType.DMA,
    ],
)
def _topk_sc(x_hbm, out_hbm, row_a, row_b, fine_v, gsum_v, ci_v,
             sem_ia, sem_ib, sem_oa, sem_ob):
    _topk_body(x_hbm, out_hbm, row_a, row_b, fine_v, gsum_v, ci_v,
               sem_ia, sem_ib, sem_oa, sem_ob)


def kernel(x):
    return _topk_sc(x)


# speedup vs baseline: 1.7509x; 1.1021x over previous
import functools

import jax
import jax.numpy as jnp
from jax import lax
from jax.experimental import pallas as pl
from jax.experimental.pallas import tpu as pltpu
from jax.experimental.pallas import tpu_sc as plsc

ROWS = 128
N = 32768
KK = 256
L = 16
NCHUNK = N // L
NWORKERS = 32
ROWS_PER_W = ROWS // NWORKERS
FBINS = 1024
NGROUPS = FBINS // L

MIN32 = -(2 ** 31)
M7F = 0x7FFFFFFF
LOW22 = 0x3FFFFF


def _key_of(v):
    b = lax.bitcast_convert_type(v, jnp.int32)
    return b ^ ((b >> 31) & M7F)


def _topk_body(x_hbm, out_hbm, row_v, fine_v, gsum_v, ck_v, ci_v):
    cid = lax.axis_index("c")
    sid = lax.axis_index("s")
    wid = sid * 2 + cid

    zeros16 = jnp.zeros((L,), jnp.int32)
    ones16 = jnp.ones((L,), jnp.int32)
    iota16 = lax.iota(jnp.int32, L)
    lane_off = iota16 * FBINS
    lane15 = iota16 == (L - 1)

    def per_row(r, _carry):
        row = wid * ROWS_PER_W + r
        pltpu.sync_copy(x_hbm.at[row], row_v)

        @plsc.parallel_loop(0, FBINS, unroll=8)
        def _clear(i):
            fine_v[pl.ds(pl.multiple_of(i * L, L), L)] = zeros16

        @plsc.parallel_loop(0, NCHUNK, unroll=8)
        def _hist(i):
            v = row_v[pl.ds(pl.multiple_of(i * L, L), L)]
            key = _key_of(v)
            ubin = ((key ^ MIN32) >> 22) & (FBINS - 1)
            plsc.addupdate_scatter(fine_v, [lane_off + ubin], ones16)

        @plsc.parallel_loop(0, NGROUPS, unroll=2)
        def _gsum(g):
            base = g * L
            vec = fine_v[pl.ds(pl.multiple_of(base, L), L)]
            for l in range(1, L):
                vec = vec + fine_v[pl.ds(pl.multiple_of(l * FBINS + base, L), L)]
            cum = plsc.cumsum(vec)
            gidx = jnp.full((L,), g, jnp.int32)
            plsc.store_scatter(gsum_v, [gidx], cum, mask=lane15)

        gvecs = [gsum_v[pl.ds(k * L, L)] for k in range(NGROUPS // L)]
        sufs = [jnp.flip(plsc.cumsum(jnp.flip(g, 0)), 0) for g in gvecs]
        tots = [s[0] for s in sufs]
        cnt_g = zeros16
        hi = jnp.int32(0)
        for k in range(NGROUPS // L - 1, -1, -1):
            cnt_g = cnt_g + plsc.all_reduce_population_count((sufs[k] + hi) >= KK)
            hi = hi + tots[k]
        gstar = jnp.max(cnt_g) - 1
        acc_above = jnp.int32(0)
        for k in range(NGROUPS // L):
            acc_above = acc_above + jnp.sum(
                jnp.where((k * L + iota16) > gstar, gvecs[k], 0))

        base = gstar * L
        vec = fine_v[pl.ds(pl.multiple_of(base, L), L)]
        for l in range(1, L):
            vec = vec + fine_v[pl.ds(pl.multiple_of(l * FBINS + base, L), L)]
        suf = jnp.flip(plsc.cumsum(jnp.flip(vec, 0)), 0)
        cond = (acc_above + suf) >= KK
        cntv = plsc.all_reduce_population_count(cond)
        posv = cntv - 1
        b1v = gstar * L + posv
        count_above = acc_above + jnp.sum(jnp.where(iota16 > posv, vec, 0))
        k_rem = jnp.int32(KK) - count_above
        b1s = gstar * L + jnp.max(cntv) - 1

        @plsc.parallel_loop(0, NCHUNK, unroll=8, carry=zeros16)
        def _comp(i, nc_vec):
            v = row_v[pl.ds(pl.multiple_of(i * L, L), L)]
            key = _key_of(v)
            ux = key ^ MIN32
            ubin = (ux >> 22) & (FBINS - 1)
            m = ubin == b1v
            pref = plsc.cumsum(m.astype(jnp.int32))
            pos = nc_vec + pref - 1
            plsc.store_scatter(ck_v, [pos], ux & LOW22, mask=m)
            plsc.store_scatter(ci_v, [pos], i * L + iota16, mask=m)
            return nc_vec + plsc.all_reduce_population_count(m)

        nc = jnp.max(_comp)
        nch = (nc + (L - 1)) // L

        def refine(bi, prefix):
            bit = jnp.int32(21) - bi
            cand = prefix | (jnp.int32(1) << bit)

            def cnt_chunk(j, cnt_v):
                low = ck_v[pl.ds(pl.multiple_of(j * L, L), L)]
                valid = (j * L + iota16) < nc
                m = jnp.logical_and(low >= cand, valid)
                return cnt_v + plsc.all_reduce_population_count(m)

            cnt = jnp.max(lax.fori_loop(0, nch, cnt_chunk, zeros16))
            return jnp.where(cnt >= k_rem, cand, prefix)

        prefix = lax.fori_loop(0, 22, refine, jnp.int32(0))

        def gt_chunk(j, cnt_v):
            low = ck_v[pl.ds(pl.multiple_of(j * L, L), L)]
            valid = (j * L + iota16) < nc
            m = jnp.logical_and(low > prefix, valid)
            return cnt_v + plsc.all_reduce_population_count(m)

        c_gt = jnp.max(lax.fori_loop(0, nch, gt_chunk, zeros16))
        ties_needed = k_rem - c_gt

        t_u = (b1s << 22) | prefix
        t_s = t_u ^ MIN32
        tb_vec = jnp.full((L,), t_s, jnp.int32)
        tb_vec = tb_vec ^ ((tb_vec >> 31) & M7F)
        t_f_vec = lax.bitcast_convert_type(tb_vec, jnp.float32)
        t_f = t_f_vec[0]

        @plsc.parallel_loop(0, NCHUNK, unroll=8)
        def _outp(i):
            sl = pl.ds(pl.multiple_of(i * L, L), L)
            v = row_v[sl]
            row_v[sl] = jnp.where(v >= t_f, v, jnp.float32(0.0))

        def fixup(j, seq_vec):
            sl = pl.ds(pl.multiple_of(j * L, L), L)
            low = ck_v[sl]
            idx = ci_v[sl]
            valid = (j * L + iota16) < nc
            m_eq = jnp.logical_and(low == prefix, valid)
            pref = plsc.cumsum(m_eq.astype(jnp.int32))
            drop = jnp.logical_and(m_eq, (seq_vec + pref) > ties_needed)
            plsc.store_scatter(row_v, [idx], jnp.zeros((L,), jnp.float32),
                               mask=drop)
            return seq_vec + plsc.all_reduce_population_count(m_eq)

        lax.fori_loop(0, nch, fixup, zeros16)

        pltpu.sync_copy(row_v, out_hbm.at[row])
        return _carry

    lax.fori_loop(0, ROWS_PER_W, per_row, 0)


@functools.partial(
    pl.kernel,
    out_type=jax.ShapeDtypeStruct((ROWS, N), jnp.float32),
    mesh=plsc.VectorSubcoreMesh(core_axis_name="c", subcore_axis_name="s"),
    compiler_params=pltpu.CompilerParams(needs_layout_passes=False),
    scratch_types=[
        pltpu.VMEM((N,), jnp.float32),
        pltpu.VMEM((L * FBINS,), jnp.int32),
        pltpu.VMEM((NGROUPS,), jnp.int32),
        pltpu.VMEM((N,), jnp.int32),
        pltpu.VMEM((N,), jnp.int32),
    ],
)
def _topk_sc(x_hbm, out_hbm, row_v, fine_v, gsum_v, ck_v, ci_v):
    _topk_body(x_hbm, out_hbm, row_v, fine_v, gsum_v, ck_v, ci_v)


def kernel(x):
    return _topk_sc(x)


# speedup vs baseline: 1.7908x; 1.0228x over previous
import functools

import jax
import jax.numpy as jnp
from jax import lax
from jax.experimental import pallas as pl
from jax.experimental.pallas import tpu as pltpu
from jax.experimental.pallas import tpu_sc as plsc

ROWS = 128
N = 32768
KK = 256
L = 16
NCHUNK = N // L
NWORKERS = 32
ROWS_PER_W = ROWS // NWORKERS
FBINS = 1024
NGROUPS = FBINS // L

MIN32 = -(2 ** 31)
M7F = 0x7FFFFFFF
LOW22 = 0x3FFFFF


def _key_of(v):
    b = lax.bitcast_convert_type(v, jnp.int32)
    return b ^ ((b >> 31) & M7F)


def _topk_body(x_hbm, out_hbm, row_v, fine_v, gsum_v, ck_v, ci_v):
    cid = lax.axis_index("c")
    sid = lax.axis_index("s")
    wid = sid * 2 + cid

    zeros16 = jnp.zeros((L,), jnp.int32)
    ones16 = jnp.ones((L,), jnp.int32)
    iota16 = lax.iota(jnp.int32, L)
    lane_off = iota16 * FBINS
    lane15 = iota16 == (L - 1)

    def per_row(r, _carry):
        row = wid * ROWS_PER_W + r
        pltpu.sync_copy(x_hbm.at[row], row_v)

        @plsc.parallel_loop(0, FBINS, unroll=8)
        def _clear(i):
            fine_v[pl.ds(pl.multiple_of(i * L, L), L)] = zeros16

        @plsc.parallel_loop(0, NCHUNK, unroll=8)
        def _hist(i):
            v = row_v[pl.ds(pl.multiple_of(i * L, L), L)]
            b = lax.bitcast_convert_type(v, jnp.int32)
            ux = b ^ ((b >> 31) | MIN32)
            ubin = (ux >> 22) & (FBINS - 1)
            plsc.addupdate_scatter(fine_v, [lane_off + ubin], ones16)

        @plsc.parallel_loop(0, NGROUPS, unroll=2)
        def _gsum(g):
            base = g * L
            vec = fine_v[pl.ds(pl.multiple_of(base, L), L)]
            for l in range(1, L):
                vec = vec + fine_v[pl.ds(pl.multiple_of(l * FBINS + base, L), L)]
            cum = plsc.cumsum(vec)
            gidx = jnp.full((L,), g, jnp.int32)
            plsc.store_scatter(gsum_v, [gidx], cum, mask=lane15)

        gvecs = [gsum_v[pl.ds(k * L, L)] for k in range(NGROUPS // L)]
        sufs = [jnp.flip(plsc.cumsum(jnp.flip(g, 0)), 0) for g in gvecs]
        tots = [s[0] for s in sufs]
        cnt_g = zeros16
        hi = jnp.int32(0)
        for k in range(NGROUPS // L - 1, -1, -1):
            cnt_g = cnt_g + plsc.all_reduce_population_count((sufs[k] + hi) >= KK)
            hi = hi + tots[k]
        gstar = jnp.max(cnt_g) - 1
        acc_above = jnp.int32(0)
        for k in range(NGROUPS // L):
            acc_above = acc_above + jnp.sum(
                jnp.where((k * L + iota16) > gstar, gvecs[k], 0))

        base = gstar * L
        vec = fine_v[pl.ds(pl.multiple_of(base, L), L)]
        for l in range(1, L):
            vec = vec + fine_v[pl.ds(pl.multiple_of(l * FBINS + base, L), L)]
        suf = jnp.flip(plsc.cumsum(jnp.flip(vec, 0)), 0)
        cond = (acc_above + suf) >= KK
        cntv = plsc.all_reduce_population_count(cond)
        posv = cntv - 1
        b1v = gstar * L + posv
        count_above = acc_above + jnp.sum(jnp.where(iota16 > posv, vec, 0))
        k_rem = jnp.int32(KK) - count_above
        b1s = gstar * L + jnp.max(cntv) - 1

        @plsc.parallel_loop(0, NCHUNK, unroll=8, carry=zeros16)
        def _comp(i, nc_vec):
            v = row_v[pl.ds(pl.multiple_of(i * L, L), L)]
            b = lax.bitcast_convert_type(v, jnp.int32)
            ux = b ^ ((b >> 31) | MIN32)
            ubin = (ux >> 22) & (FBINS - 1)
            m = ubin == b1v
            pref = plsc.cumsum(m.astype(jnp.int32))
            pos = nc_vec + pref - 1
            plsc.store_scatter(ck_v, [pos], ux & LOW22, mask=m)
            plsc.store_scatter(ci_v, [pos], i * L + iota16, mask=m)
            return nc_vec + plsc.all_reduce_population_count(m)

        nc = jnp.max(_comp)
        nch = (nc + (L - 1)) // L

        def refine(bi, prefix):
            bit = jnp.int32(21) - bi
            cand = prefix | (jnp.int32(1) << bit)

            def cnt_chunk(j, cnt_v):
                low = ck_v[pl.ds(pl.multiple_of(j * L, L), L)]
                valid = (j * L + iota16) < nc
                m = jnp.logical_and(low >= cand, valid)
                return cnt_v + plsc.all_reduce_population_count(m)

            cnt = jnp.max(lax.fori_loop(0, nch, cnt_chunk, zeros16))
            return jnp.where(cnt >= k_rem, cand, prefix)

        prefix = lax.fori_loop(0, 22, refine, jnp.int32(0))

        def gt_chunk(j, cnt_v):
            low = ck_v[pl.ds(pl.multiple_of(j * L, L), L)]
            valid = (j * L + iota16) < nc
            m = jnp.logical_and(low > prefix, valid)
            return cnt_v + plsc.all_reduce_population_count(m)

        c_gt = jnp.max(lax.fori_loop(0, nch, gt_chunk, zeros16))
        ties_needed = k_rem - c_gt

        t_u = (b1s << 22) | prefix
        t_s = t_u ^ MIN32
        tb_vec = jnp.full((L,), t_s, jnp.int32)
        tb_vec = tb_vec ^ ((tb_vec >> 31) & M7F)
        t_f_vec = lax.bitcast_convert_type(tb_vec, jnp.float32)
        t_f = t_f_vec[0]

        @plsc.parallel_loop(0, NCHUNK, unroll=8)
        def _outp(i):
            sl = pl.ds(pl.multiple_of(i * L, L), L)
            v = row_v[sl]
            row_v[sl] = jnp.where(v >= t_f, v, jnp.float32(0.0))

        def fixup(j, seq_vec):
            sl = pl.ds(pl.multiple_of(j * L, L), L)
            low = ck_v[sl]
            idx = ci_v[sl]
            valid = (j * L + iota16) < nc
            m_eq = jnp.logical_and(low == prefix, valid)
            pref = plsc.cumsum(m_eq.astype(jnp.int32))
            drop = jnp.logical_and(m_eq, (seq_vec + pref) > ties_needed)
            plsc.store_scatter(row_v, [idx], jnp.zeros((L,), jnp.float32),
                               mask=drop)
            return seq_vec + plsc.all_reduce_population_count(m_eq)

        lax.fori_loop(0, nch, fixup, zeros16)

        pltpu.sync_copy(row_v, out_hbm.at[row])
        return _carry

    lax.fori_loop(0, ROWS_PER_W, per_row, 0)


@functools.partial(
    pl.kernel,
    out_type=jax.ShapeDtypeStruct((ROWS, N), jnp.float32),
    mesh=plsc.VectorSubcoreMesh(core_axis_name="c", subcore_axis_name="s"),
    compiler_params=pltpu.CompilerParams(needs_layout_passes=False),
    scratch_types=[
        pltpu.VMEM((N,), jnp.float32),
        pltpu.VMEM((L * FBINS,), jnp.int32),
        pltpu.VMEM((NGROUPS,), jnp.int32),
        pltpu.VMEM((N,), jnp.int32),
        pltpu.VMEM((N,), jnp.int32),
    ],
)
def _topk_sc(x_hbm, out_hbm, row_v, fine_v, gsum_v, ck_v, ci_v):
    _topk_body(x_hbm, out_hbm, row_v, fine_v, gsum_v, ck_v, ci_v)


def kernel(x):
    return _topk_sc(x)
